# baseline (speedup 1.0000x reference)
import functools

import jax
import jax.numpy as jnp
from jax import lax
from jax.experimental import pallas as pl
from jax.experimental.pallas import tpu as pltpu
from jax.experimental.pallas import tpu_sc as plsc

_B = 16384
_NW = 32
_SPW = _B // _NW
_S = 8
_NCHUNK = _SPW // _S
_L = 50
_DD = 48
_DP = 32


def _augment(table, W1, b1, W2, stored):
    V, D = table.shape
    H = W1.shape[1]
    blk = 2000
    b1r = b1.reshape(1, H)
    w2r = W2.reshape(1, H)

    def body(t_ref, w1_ref, b1_ref, w2_ref, o_ref):
        t = t_ref[...]
        h = jnp.tanh(t @ w1_ref[...] + b1_ref[...])
        s = jnp.sum(h * w2_ref[...], axis=1, keepdims=True)
        shift = jnp.sum(jnp.abs(w2_ref[...]))
        p = jnp.exp(s - shift)
        pt = t * p
        pcol = jnp.concatenate([p, jnp.zeros((blk, 15), jnp.float32)], axis=1)
        if D == 32:
            groups = [(pt[:, 0:16], pt[:, 16:32]),
                      (pcol, jnp.zeros((blk, 16), jnp.float32))]
        else:
            groups = [(pt, pcol)]
        pieces = [jnp.stack([e, o], axis=-1).reshape(blk, 32)
                  for e, o in groups]
        o_ref[...] = jnp.concatenate(pieces, axis=1).astype(jnp.bfloat16)

    return pl.pallas_call(
        body,
        grid=(V // blk,),
        in_specs=[
            pl.BlockSpec((blk, D), lambda i: (i, 0)),
            pl.BlockSpec((D, H), lambda i: (0, 0)),
            pl.BlockSpec((1, H), lambda i: (0, 0)),
            pl.BlockSpec((1, H), lambda i: (0, 0)),
        ],
        out_specs=pl.BlockSpec((blk, stored), lambda i: (i, 0)),
        out_shape=jax.ShapeDtypeStruct((V, stored), jnp.bfloat16),
    )(table, W1, b1r, w2r)


def _sc_pool(aug, idx_c, idx_p, nch):
    mesh = plsc.VectorSubcoreMesh(core_axis_name="c", subcore_axis_name="s")
    stored = aug.shape[1]
    width = nch * 16
    out_t = (
        jax.ShapeDtypeStruct((_B, width), jnp.float32),
        jax.ShapeDtypeStruct((_B, width), jnp.float32),
    )

    @functools.partial(
        pl.kernel,
        mesh=mesh,
        out_type=out_t,
        compiler_params=pltpu.CompilerParams(use_tc_tiling_on_sc=False,
                                             needs_layout_passes=False),
        scratch_types=[
            pltpu.VMEM((2, _SPW, _L), jnp.int32),
            pltpu.VMEM((2, _S * _L, stored), jnp.bfloat16),
            pltpu.VMEM((2, _S, width), jnp.float32),
            pltpu.SemaphoreType.DMA,
            pltpu.SemaphoreType.DMA,
            pltpu.SemaphoreType.DMA,
            pltpu.SemaphoreType.DMA,
        ],
    )
    def k(tab_hbm, ci_hbm, pi_hbm, oc, op, idx_v, rows_b, stage,
          sem0, sem1, osem0, osem1):
        wid = lax.axis_index("s") * 2 + lax.axis_index("c")
        sems = (sem0, sem1)
        osems = (osem0, osem1)

        icp = pltpu.async_copy(ci_hbm.at[pl.ds(wid * _SPW, _SPW)],
                               idx_v.at[0], sem0)
        ipp = pltpu.async_copy(pi_hbm.at[pl.ds(wid * _SPW, _SPW)],
                               idx_v.at[1], sem1)
        icp.wait()
        ipp.wait()

        def run_field(f, out_hbm):
            idx_slab = idx_v.at[f]

            def fire(b, c):
                for j in range(_S):
                    pltpu.async_copy(tab_hbm.at[idx_slab.at[c * _S + j]],
                                     rows_b.at[b].at[pl.ds(j * _L, _L)],
                                     sems[b])

            def wait_rows(b):
                pltpu.make_async_copy(tab_hbm.at[pl.ds(0, _S * _L)],
                                      rows_b.at[b], sems[b]).wait()

            def wait_out(b):
                pltpu.make_async_copy(stage.at[b],
                                      out_hbm.at[pl.ds(0, _S)],
                                      osems[b]).wait()

            def row_chunks(rows_v, r):
                chunks = []
                for g in range(stored // 32):
                    v = plsc.bitcast(rows_v[r, pl.ds(g * 32, 32)], jnp.int32)
                    chunks.append(
                        plsc.bitcast(jnp.left_shift(v, 16), jnp.float32))
                    if len(chunks) < nch:
                        chunks.append(plsc.bitcast(
                            jnp.bitwise_and(v, jnp.int32(-65536)),
                            jnp.float32))
                return tuple(chunks[:nch])

            def accumulate(b, c):
                rows_v = rows_b.at[b]
                st = stage.at[b]

                @pl.loop(0, _S)
                def _sample(s):
                    base = s * _L

                    def body(i, accs):
                        return tuple(
                            a + ch
                            for a, ch in zip(accs,
                                             row_chunks(rows_v, base + i))
                        )

                    init = row_chunks(rows_v, base)
                    accs = lax.fori_loop(1, _L, body, init, unroll=7)
                    for ch in range(nch):
                        st[s, pl.ds(ch * 16, 16)] = accs[ch]

                pltpu.async_copy(st,
                                 out_hbm.at[pl.ds(wid * _SPW + c * _S, _S)],
                                 osems[b])

            fire(0, 0)
            fire(1, 1)

            @pl.loop(0, _NCHUNK, step=2)
            def _chunk(c):
                for b in range(2):
                    wait_rows(b)

                    @pl.when(c >= 2)
                    def _():
                        wait_out(b)

                    accumulate(b, c + b)

                    @pl.when(c + b + 2 < _NCHUNK)
                    def _():
                        fire(b, c + b + 2)

            wait_out(0)
            wait_out(1)

        run_field(0, oc)
        run_field(1, op)

    return k(aug, idx_c, idx_p)


def _sub_pool(c_idx, p_idx, st_pad, sW1, sb1, sW2):
    blk = 2048
    K = c_idx.shape[1]
    V = st_pad.shape[0]
    H = sW1.shape[1]
    b1r = sb1.reshape(1, H)
    w2r = sW2.reshape(1, H)

    def body(ci_ref, pi_ref, t_ref, w1_ref, b1_ref, w2_ref, o_ref):
        t = t_ref[...]
        h = jnp.tanh(t @ w1_ref[...] + b1_ref[...])
        s = jnp.sum(h * w2_ref[...], axis=1, keepdims=True)
        shift = jnp.sum(jnp.abs(w2_ref[...]))
        p = jnp.exp(s - shift)
        pt = t * p
        vid = lax.broadcasted_iota(jnp.int32, (blk, V), 1)

        def pool(idx):
            cnt = jnp.zeros((blk, V), jnp.float32)
            for i in range(K):
                cnt = cnt + (idx[:, i:i + 1] == vid).astype(jnp.float32)
            num = cnt @ pt
            den = cnt @ p
            return num / den

        o_ref[...] = jnp.concatenate(
            [pool(ci_ref[...]), pool(pi_ref[...])], axis=1)

    return pl.pallas_call(
        body,
        grid=(_B // blk,),
        in_specs=[
            pl.BlockSpec((blk, K), lambda i: (i, 0)),
            pl.BlockSpec((blk, K), lambda i: (i, 0)),
            pl.BlockSpec((V, 16), lambda i: (0, 0)),
            pl.BlockSpec((16, H), lambda i: (0, 0)),
            pl.BlockSpec((1, H), lambda i: (0, 0)),
            pl.BlockSpec((1, H), lambda i: (0, 0)),
        ],
        out_specs=pl.BlockSpec((blk, 32), lambda i: (i, 0)),
        out_shape=jax.ShapeDtypeStruct((_B, 32), jnp.float32),
    )(c_idx, p_idx, st_pad, sW1, b1r, w2r)


def _final(cd_s, cp_s, pd_s, pp_s, sub, fc1_W, fc1_b, fc2_W, fc2_b,
           fc3_W, fc3_b):
    blk = 1024

    def body(cd_ref, cp_ref, pd_ref, pp_ref, sub_ref,
             w1_ref, b1_ref, w2_ref, b2_ref, w3_ref, b3_ref, o_ref):
        cd = cd_ref[...]
        cp = cp_ref[...]
        pd = pd_ref[...]
        pp = pp_ref[...]
        sb = sub_ref[...]
        x = jnp.concatenate([
            cd[:, 0:32] / cd[:, 32:33],
            cp[:, 0:16] / cp[:, 16:17],
            sb[:, 0:16],
            pd[:, 0:32] / pd[:, 32:33],
            pp[:, 0:16] / pp[:, 16:17],
            sb[:, 16:32],
        ], axis=1)
        h = x @ w1_ref[...] + b1_ref[...]
        h = jnp.where(h >= 0, h, 0.01 * h)
        h = h @ w2_ref[...] + b2_ref[...]
        h = jnp.where(h >= 0, h, 0.01 * h)
        o_ref[...] = h @ w3_ref[...] + b3_ref[...]

    return pl.pallas_call(
        body,
        grid=(_B // blk,),
        in_specs=[
            pl.BlockSpec((blk, _DD), lambda i: (i, 0)),
            pl.BlockSpec((blk, _DP), lambda i: (i, 0)),
            pl.BlockSpec((blk, _DD), lambda i: (i, 0)),
            pl.BlockSpec((blk, _DP), lambda i: (i, 0)),
            pl.BlockSpec((blk, 32), lambda i: (i, 0)),
            pl.BlockSpec((128, 128), lambda i: (0, 0)),
            pl.BlockSpec((1, 128), lambda i: (0, 0)),
            pl.BlockSpec((128, 64), lambda i: (0, 0)),
            pl.BlockSpec((1, 64), lambda i: (0, 0)),
            pl.BlockSpec((64, 1), lambda i: (0, 0)),
            pl.BlockSpec((1, 1), lambda i: (0, 0)),
        ],
        out_specs=pl.BlockSpec((blk, 1), lambda i: (i, 0)),
        out_shape=jax.ShapeDtypeStruct((_B, 1), jnp.float32),
    )(cd_s, cp_s, pd_s, pp_s, sub,
      fc1_W, fc1_b.reshape(1, 128), fc2_W, fc2_b.reshape(1, 64),
      fc3_W, fc3_b.reshape(1, 1))


def kernel(compound_diseases, compound_phenotypes,
           compound_subcellular_locations, protein_diseases,
           protein_phenotypes, protein_subcellular_locations,
           disease_table, phenotype_table, subcellular_table,
           dW1, db1, dW2, db2, pW1, pb1, pW2, pb2, sW1, sb1, sW2, sb2,
           fc1_W, fc1_b, fc2_W, fc2_b, fc3_W, fc3_b):
    dis_aug = _augment(disease_table, dW1, db1, dW2, 64)
    cd_s, pd_s = _sc_pool(dis_aug, compound_diseases, protein_diseases, 3)

    phe_aug = _augment(phenotype_table, pW1, pb1, pW2, 32)
    cp_s, pp_s = _sc_pool(phe_aug, compound_phenotypes, protein_phenotypes, 2)

    st_pad = jnp.pad(subcellular_table, ((0, 2), (0, 0)))
    sub = _sub_pool(compound_subcellular_locations,
                    protein_subcellular_locations, st_pad, sW1, sb1, sW2)

    return _final(cd_s, cp_s, pd_s, pp_s, sub,
                  fc1_W, fc1_b, fc2_W, fc2_b, fc3_W, fc3_b)


# speedup vs baseline: 3.6506x; 3.6506x over previous
import functools

import jax
import jax.numpy as jnp
from jax import lax
from jax.experimental import pallas as pl
from jax.experimental.pallas import tpu as pltpu
from jax.experimental.pallas import tpu_sc as plsc

_B = 16384
_NW = 32
_SPW = _B // _NW
_S = 8
_NCHUNK = _SPW // _S
_L = 50
_DD = 48
_DP = 32


def _augment(table, W1, b1, W2, stored):
    V, D = table.shape
    H = W1.shape[1]
    blk = 2000
    b1r = b1.reshape(1, H)
    w2r = W2.reshape(1, H)

    def rnd(u):
        return u + jnp.int32(0x7FFF) + (lax.shift_right_logical(u, 16)
                                        & jnp.int32(1))

    def pack(e, o):
        ue = lax.bitcast_convert_type(e, jnp.int32)
        uo = lax.bitcast_convert_type(o, jnp.int32)
        lo = lax.shift_right_logical(rnd(ue), 16)
        hi = rnd(uo) & jnp.int32(-65536)
        return lo | hi

    def body(t_ref, w1_ref, b1_ref, w2_ref, o_ref):
        t = t_ref[...]
        h = jnp.tanh(t @ w1_ref[...] + b1_ref[...])
        s = jnp.sum(h * w2_ref[...], axis=1, keepdims=True)
        shift = jnp.sum(jnp.abs(w2_ref[...]))
        p = jnp.exp(s - shift)
        pt = t * p
        pcol = jnp.concatenate([p, jnp.zeros((blk, 15), jnp.float32)], axis=1)
        if D == 32:
            o_ref[:, 0:16] = pack(pt[:, 0:16], pt[:, 16:32])
            o_ref[:, 16:32] = pack(pcol, jnp.zeros((blk, 16), jnp.float32))
        else:
            o_ref[...] = pack(pt, pcol)

    return pl.pallas_call(
        body,
        grid=(V // blk,),
        in_specs=[
            pl.BlockSpec((blk, D), lambda i: (i, 0)),
            pl.BlockSpec((D, H), lambda i: (0, 0)),
            pl.BlockSpec((1, H), lambda i: (0, 0)),
            pl.BlockSpec((1, H), lambda i: (0, 0)),
        ],
        out_specs=pl.BlockSpec((blk, stored), lambda i: (i, 0)),
        out_shape=jax.ShapeDtypeStruct((V, stored), jnp.int32),
    )(table, W1, b1r, w2r)


def _sc_pool(aug, idx_c, idx_p, nch):
    mesh = plsc.VectorSubcoreMesh(core_axis_name="c", subcore_axis_name="s")
    stored = aug.shape[1]
    width = nch * 16
    out_t = (
        jax.ShapeDtypeStruct((_B, width), jnp.float32),
        jax.ShapeDtypeStruct((_B, width), jnp.float32),
    )

    @functools.partial(
        pl.kernel,
        mesh=mesh,
        out_type=out_t,
        compiler_params=pltpu.CompilerParams(use_tc_tiling_on_sc=False,
                                             needs_layout_passes=False),
        scratch_types=[
            pltpu.VMEM((2, _SPW, _L), jnp.int32),
            pltpu.VMEM((2, _S * _L, stored), jnp.int32),
            pltpu.VMEM((2, _S, width), jnp.float32),
            pltpu.SemaphoreType.DMA,
            pltpu.SemaphoreType.DMA,
            pltpu.SemaphoreType.DMA,
            pltpu.SemaphoreType.DMA,
        ],
    )
    def k(tab_hbm, ci_hbm, pi_hbm, oc, op, idx_v, rows_b, stage,
          sem0, sem1, osem0, osem1):
        wid = lax.axis_index("s") * 2 + lax.axis_index("c")
        sems = (sem0, sem1)
        osems = (osem0, osem1)

        icp = pltpu.async_copy(ci_hbm.at[pl.ds(wid * _SPW, _SPW)],
                               idx_v.at[0], sem0)
        ipp = pltpu.async_copy(pi_hbm.at[pl.ds(wid * _SPW, _SPW)],
                               idx_v.at[1], sem1)
        icp.wait()
        ipp.wait()

        def run_field(f, out_hbm):
            idx_slab = idx_v.at[f]

            def fire(b, c):
                for j in range(_S):
                    pltpu.async_copy(tab_hbm.at[idx_slab.at[c * _S + j]],
                                     rows_b.at[b].at[pl.ds(j * _L, _L)],
                                     sems[b])

            def wait_rows(b):
                pltpu.make_async_copy(tab_hbm.at[pl.ds(0, _S * _L)],
                                      rows_b.at[b], sems[b]).wait()

            def wait_out(b):
                pltpu.make_async_copy(stage.at[b],
                                      out_hbm.at[pl.ds(0, _S)],
                                      osems[b]).wait()

            def row_chunks(rows_v, r):
                chunks = []
                for g in range(stored // 16):
                    v = rows_v[r, pl.ds(g * 16, 16)]
                    chunks.append(
                        plsc.bitcast(jnp.left_shift(v, 16), jnp.float32))
                    if len(chunks) < nch:
                        chunks.append(plsc.bitcast(
                            jnp.bitwise_and(v, jnp.int32(-65536)),
                            jnp.float32))
                return tuple(chunks[:nch])

            def accumulate(b, c):
                rows_v = rows_b.at[b]
                st = stage.at[b]

                @pl.loop(0, _S)
                def _sample(s):
                    base = s * _L

                    def body(i, accs):
                        return tuple(
                            a + ch
                            for a, ch in zip(accs,
                                             row_chunks(rows_v, base + i))
                        )

                    init = row_chunks(rows_v, base)
                    accs = lax.fori_loop(1, _L, body, init, unroll=7)
                    for ch in range(nch):
                        st[s, pl.ds(ch * 16, 16)] = accs[ch]

                pltpu.async_copy(st,
                                 out_hbm.at[pl.ds(wid * _SPW + c * _S, _S)],
                                 osems[b])

            fire(0, 0)
            fire(1, 1)

            @pl.loop(0, _NCHUNK, step=2)
            def _chunk(c):
                for b in range(2):
                    wait_rows(b)

                    @pl.when(c >= 2)
                    def _():
                        wait_out(b)

                    accumulate(b, c + b)

                    @pl.when(c + b + 2 < _NCHUNK)
                    def _():
                        fire(b, c + b + 2)

            wait_out(0)
            wait_out(1)

        run_field(0, oc)
        run_field(1, op)

    return k(aug, idx_c, idx_p)


def _sub_pool(c_idx, p_idx, st_pad, sW1, sb1, sW2):
    blk = 2048
    K = c_idx.shape[1]
    V = st_pad.shape[0]
    H = sW1.shape[1]
    b1r = sb1.reshape(1, H)
    w2r = sW2.reshape(1, H)

    def body(ci_ref, pi_ref, t_ref, w1_ref, b1_ref, w2_ref, o_ref):
        t = t_ref[...]
        h = jnp.tanh(t @ w1_ref[...] + b1_ref[...])
        s = jnp.sum(h * w2_ref[...], axis=1, keepdims=True)
        shift = jnp.sum(jnp.abs(w2_ref[...]))
        p = jnp.exp(s - shift)
        pt = t * p
        vid = lax.broadcasted_iota(jnp.int32, (blk, V), 1)

        def pool(idx):
            cnt = jnp.zeros((blk, V), jnp.float32)
            for i in range(K):
                cnt = cnt + (idx[:, i:i + 1] == vid).astype(jnp.float32)
            num = cnt @ pt
            den = cnt @ p
            return num / den

        o_ref[...] = jnp.concatenate(
            [pool(ci_ref[...]), pool(pi_ref[...])], axis=1)

    return pl.pallas_call(
        body,
        grid=(_B // blk,),
        in_specs=[
            pl.BlockSpec((blk, K), lambda i: (i, 0)),
            pl.BlockSpec((blk, K), lambda i: (i, 0)),
            pl.BlockSpec((V, 16), lambda i: (0, 0)),
            pl.BlockSpec((16, H), lambda i: (0, 0)),
            pl.BlockSpec((1, H), lambda i: (0, 0)),
            pl.BlockSpec((1, H), lambda i: (0, 0)),
        ],
        out_specs=pl.BlockSpec((blk, 32), lambda i: (i, 0)),
        out_shape=jax.ShapeDtypeStruct((_B, 32), jnp.float32),
    )(c_idx, p_idx, st_pad, sW1, b1r, w2r)


def _final(cd_s, cp_s, pd_s, pp_s, sub, fc1_W, fc1_b, fc2_W, fc2_b,
           fc3_W, fc3_b):
    blk = 1024

    def body(cd_ref, cp_ref, pd_ref, pp_ref, sub_ref,
             w1_ref, b1_ref, w2_ref, b2_ref, w3_ref, b3_ref, o_ref):
        cd = cd_ref[...]
        cp = cp_ref[...]
        pd = pd_ref[...]
        pp = pp_ref[...]
        sb = sub_ref[...]
        x = jnp.concatenate([
            cd[:, 0:32] / cd[:, 32:33],
            cp[:, 0:16] / cp[:, 16:17],
            sb[:, 0:16],
            pd[:, 0:32] / pd[:, 32:33],
            pp[:, 0:16] / pp[:, 16:17],
            sb[:, 16:32],
        ], axis=1)
        h = x @ w1_ref[...] + b1_ref[...]
        h = jnp.where(h >= 0, h, 0.01 * h)
        h = h @ w2_ref[...] + b2_ref[...]
        h = jnp.where(h >= 0, h, 0.01 * h)
        o_ref[...] = h @ w3_ref[...] + b3_ref[...]

    return pl.pallas_call(
        body,
        grid=(_B // blk,),
        in_specs=[
            pl.BlockSpec((blk, _DD), lambda i: (i, 0)),
            pl.BlockSpec((blk, _DP), lambda i: (i, 0)),
            pl.BlockSpec((blk, _DD), lambda i: (i, 0)),
            pl.BlockSpec((blk, _DP), lambda i: (i, 0)),
            pl.BlockSpec((blk, 32), lambda i: (i, 0)),
            pl.BlockSpec((128, 128), lambda i: (0, 0)),
            pl.BlockSpec((1, 128), lambda i: (0, 0)),
            pl.BlockSpec((128, 64), lambda i: (0, 0)),
            pl.BlockSpec((1, 64), lambda i: (0, 0)),
            pl.BlockSpec((64, 1), lambda i: (0, 0)),
            pl.BlockSpec((1, 1), lambda i: (0, 0)),
        ],
        out_specs=pl.BlockSpec((blk, 1), lambda i: (i, 0)),
        out_shape=jax.ShapeDtypeStruct((_B, 1), jnp.float32),
    )(cd_s, cp_s, pd_s, pp_s, sub,
      fc1_W, fc1_b.reshape(1, 128), fc2_W, fc2_b.reshape(1, 64),
      fc3_W, fc3_b.reshape(1, 1))


def kernel(compound_diseases, compound_phenotypes,
           compound_subcellular_locations, protein_diseases,
           protein_phenotypes, protein_subcellular_locations,
           disease_table, phenotype_table, subcellular_table,
           dW1, db1, dW2, db2, pW1, pb1, pW2, pb2, sW1, sb1, sW2, sb2,
           fc1_W, fc1_b, fc2_W, fc2_b, fc3_W, fc3_b):
    dis_aug = _augment(disease_table, dW1, db1, dW2, 32)
    cd_s, pd_s = _sc_pool(dis_aug, compound_diseases, protein_diseases, 3)

    phe_aug = _augment(phenotype_table, pW1, pb1, pW2, 16)
    cp_s, pp_s = _sc_pool(phe_aug, compound_phenotypes, protein_phenotypes, 2)

    st_pad = jnp.pad(subcellular_table, ((0, 2), (0, 0)))
    sub = _sub_pool(compound_subcellular_locations,
                    protein_subcellular_locations, st_pad, sW1, sb1, sW2)

    return _final(cd_s, cp_s, pd_s, pp_s, sub,
                  fc1_W, fc1_b, fc2_W, fc2_b, fc3_W, fc3_b)


# speedup vs baseline: 5.0525x; 1.3840x over previous
import functools

import jax
import jax.numpy as jnp
import numpy as np
from jax import lax
from jax.experimental import pallas as pl
from jax.experimental.pallas import tpu as pltpu
from jax.experimental.pallas import tpu_sc as plsc

_B = 16384
_NW = 32
_SPW = _B // _NW
_S = 16
_NCHUNK = _SPW // _S
_L = 50
_DD = 48
_DP = 32


def _augment(table, W1, b1, W2):
    V, D = table.shape
    H = W1.shape[1]
    R = 128 // D
    V4 = V // R
    blk = 5000 if V4 % 5000 == 0 else V4
    eye = np.eye(R, dtype=np.float32)
    W1bd = jnp.kron(jnp.asarray(eye), W1)
    b1t = jnp.tile(b1, R).reshape(1, R * H)
    w2t = jnp.tile(W2.reshape(-1), R).reshape(1, R * H)
    sel = jnp.asarray(np.kron(eye, np.ones((H, 1), np.float32)))
    G = jnp.asarray(np.kron(eye, np.ones((1, D), np.float32)))
    off = 16 if D == 32 else 0
    E = np.zeros((R, 128), np.float32)
    E[np.arange(R), np.arange(R) * D + off] = 1.0
    E = jnp.asarray(E)
    if D == 32:
        P16 = np.zeros((128, 128), np.float32)
        lanes = np.arange(128)
        keep = (lanes % 32) < 16
        src = (lanes // 32) * 32 + 16 + lanes % 16
        P16[src[keep], lanes[keep]] = 1.0
        P16 = jnp.asarray(P16)
    else:
        P16 = None

    def rnd(u):
        return u + jnp.int32(0x7FFF) + (lax.shift_right_logical(u, 16)
                                        & jnp.int32(1))

    def pack(e, o):
        ue = lax.bitcast_convert_type(e, jnp.int32)
        uo = lax.bitcast_convert_type(o, jnp.int32)
        lo = lax.shift_right_logical(rnd(ue), 16)
        hi = rnd(uo) & jnp.int32(-65536)
        return lo | hi

    def body(t_ref, w1_ref, b1_ref, w2_ref, sel_ref, g_ref, e_ref, *rest):
        o_ref = rest[-1]
        t4 = t_ref[...]
        h = jnp.tanh(t4 @ w1_ref[...] + b1_ref[...])
        s = (h * w2_ref[...]) @ sel_ref[...]
        shift = jnp.sum(jnp.abs(w2_ref[...][:, 0:H]))
        p = jnp.exp(s - shift)
        pt4 = t4 * (p @ g_ref[...])
        pbig = p @ e_ref[...]
        if D == 32:
            lane = lax.broadcasted_iota(jnp.int32, (blk, 128), 1)
            lo_half = (lane & 31) < 16
            e_all = jnp.where(lo_half, pt4, pbig)
            o_all = jnp.where(lo_half, pt4 @ rest[0][...], 0.0)
        else:
            e_all = pt4
            o_all = pbig
        o_ref[...] = pack(e_all, o_all)

    full = lambda shape: pl.BlockSpec(shape, lambda i: (0, 0))
    ins = [table.reshape(V4, 128), W1bd, b1t, w2t, sel, G, E]
    specs = [pl.BlockSpec((blk, 128), lambda i: (i, 0)),
             full((128, R * H)), full((1, R * H)), full((1, R * H)),
             full((R * H, R)), full((R, 128)), full((R, 128))]
    if P16 is not None:
        ins.append(P16)
        specs.append(full((128, 128)))
    out = pl.pallas_call(
        body,
        grid=(V4 // blk,),
        in_specs=specs,
        out_specs=pl.BlockSpec((blk, 128), lambda i: (i, 0)),
        out_shape=jax.ShapeDtypeStruct((V4, 128), jnp.int32),
    )(*ins)
    return out.reshape(V, 128 // R)


def _sc_pool(aug, idx_c, idx_p, nch):
    mesh = plsc.VectorSubcoreMesh(core_axis_name="c", subcore_axis_name="s")
    stored = aug.shape[1]
    width = nch * 16
    out_t = (
        jax.ShapeDtypeStruct((_B, width), jnp.float32),
        jax.ShapeDtypeStruct((_B, width), jnp.float32),
    )

    @functools.partial(
        pl.kernel,
        mesh=mesh,
        out_type=out_t,
        compiler_params=pltpu.CompilerParams(use_tc_tiling_on_sc=False,
                                             needs_layout_passes=False),
        scratch_types=[
            pltpu.VMEM((2, _SPW, _L), jnp.int32),
            pltpu.VMEM((2, _S * _L, stored), jnp.int32),
            pltpu.VMEM((2, _S, width), jnp.float32),
            pltpu.SemaphoreType.DMA,
            pltpu.SemaphoreType.DMA,
            pltpu.SemaphoreType.DMA,
            pltpu.SemaphoreType.DMA,
        ],
    )
    def k(tab_hbm, ci_hbm, pi_hbm, oc, op, idx_v, rows_b, stage,
          sem0, sem1, osem0, osem1):
        wid = lax.axis_index("s") * 2 + lax.axis_index("c")
        sems = (sem0, sem1)
        osems = (osem0, osem1)

        icp = pltpu.async_copy(ci_hbm.at[pl.ds(wid * _SPW, _SPW)],
                               idx_v.at[0], sem0)
        ipp = pltpu.async_copy(pi_hbm.at[pl.ds(wid * _SPW, _SPW)],
                               idx_v.at[1], sem1)
        icp.wait()
        ipp.wait()

        def run_field(f, out_hbm):
            idx_slab = idx_v.at[f]

            def fire(b, c):
                for j in range(_S):
                    pltpu.async_copy(tab_hbm.at[idx_slab.at[c * _S + j]],
                                     rows_b.at[b].at[pl.ds(j * _L, _L)],
                                     sems[b])

            def wait_rows(b):
                pltpu.make_async_copy(tab_hbm.at[pl.ds(0, _S * _L)],
                                      rows_b.at[b], sems[b]).wait()

            def wait_out(b):
                pltpu.make_async_copy(stage.at[b],
                                      out_hbm.at[pl.ds(0, _S)],
                                      osems[b]).wait()

            def row_chunks(rows_v, r):
                chunks = []
                for g in range(stored // 16):
                    v = rows_v[r, pl.ds(g * 16, 16)]
                    chunks.append(
                        plsc.bitcast(jnp.left_shift(v, 16), jnp.float32))
                    if len(chunks) < nch:
                        chunks.append(plsc.bitcast(
                            jnp.bitwise_and(v, jnp.int32(-65536)),
                            jnp.float32))
                return tuple(chunks[:nch])

            def accumulate(b, c):
                rows_v = rows_b.at[b]
                st = stage.at[b]

                @pl.loop(0, _S)
                def _sample(s):
                    base = s * _L

                    def body(i, accs):
                        return tuple(
                            a + ch
                            for a, ch in zip(accs,
                                             row_chunks(rows_v, base + i))
                        )

                    init = row_chunks(rows_v, base)
                    accs = lax.fori_loop(1, _L, body, init, unroll=7)
                    for ch in range(nch):
                        st[s, pl.ds(ch * 16, 16)] = accs[ch]

                pltpu.async_copy(st,
                                 out_hbm.at[pl.ds(wid * _SPW + c * _S, _S)],
                                 osems[b])

            fire(0, 0)
            fire(1, 1)

            @pl.loop(0, _NCHUNK, step=2)
            def _chunk(c):
                for b in range(2):
                    wait_rows(b)

                    @pl.when(c >= 2)
                    def _():
                        wait_out(b)

                    accumulate(b, c + b)

                    @pl.when(c + b + 2 < _NCHUNK)
                    def _():
                        fire(b, c + b + 2)

            wait_out(0)
            wait_out(1)

        run_field(0, oc)
        run_field(1, op)

    return k(aug, idx_c, idx_p)


def _sub_pool(c_idx, p_idx, st_pad, sW1, sb1, sW2):
    blk = 2048
    K = c_idx.shape[1]
    V = st_pad.shape[0]
    H = sW1.shape[1]
    b1r = sb1.reshape(1, H)
    w2r = sW2.reshape(1, H)

    def body(ci_ref, pi_ref, t_ref, w1_ref, b1_ref, w2_ref, o_ref):
        t = t_ref[...]
        h = jnp.tanh(t @ w1_ref[...] + b1_ref[...])
        s = jnp.sum(h * w2_ref[...], axis=1, keepdims=True)
        shift = jnp.sum(jnp.abs(w2_ref[...]))
        p = jnp.exp(s - shift)
        pt = t * p
        vid = lax.broadcasted_iota(jnp.int32, (blk, V), 1)

        def pool(idx):
            cnt = jnp.zeros((blk, V), jnp.float32)
            for i in range(K):
                cnt = cnt + (idx[:, i:i + 1] == vid).astype(jnp.float32)
            num = cnt @ pt
            den = cnt @ p
            return num / den

        o_ref[...] = jnp.concatenate(
            [pool(ci_ref[...]), pool(pi_ref[...])], axis=1)

    return pl.pallas_call(
        body,
        grid=(_B // blk,),
        in_specs=[
            pl.BlockSpec((blk, K), lambda i: (i, 0)),
            pl.BlockSpec((blk, K), lambda i: (i, 0)),
            pl.BlockSpec((V, 16), lambda i: (0, 0)),
            pl.BlockSpec((16, H), lambda i: (0, 0)),
            pl.BlockSpec((1, H), lambda i: (0, 0)),
            pl.BlockSpec((1, H), lambda i: (0, 0)),
        ],
        out_specs=pl.BlockSpec((blk, 32), lambda i: (i, 0)),
        out_shape=jax.ShapeDtypeStruct((_B, 32), jnp.float32),
    )(c_idx, p_idx, st_pad, sW1, b1r, w2r)


def _final(cd_s, cp_s, pd_s, pp_s, sub, fc1_W, fc1_b, fc2_W, fc2_b,
           fc3_W, fc3_b):
    blk = 1024

    def body(cd_ref, cp_ref, pd_ref, pp_ref, sub_ref,
             w1_ref, b1_ref, w2_ref, b2_ref, w3_ref, b3_ref, o_ref):
        cd = cd_ref[...]
        cp = cp_ref[...]
        pd = pd_ref[...]
        pp = pp_ref[...]
        sb = sub_ref[...]
        x = jnp.concatenate([
            cd[:, 0:32] / cd[:, 32:33],
            cp[:, 0:16] / cp[:, 16:17],
            sb[:, 0:16],
            pd[:, 0:32] / pd[:, 32:33],
            pp[:, 0:16] / pp[:, 16:17],
            sb[:, 16:32],
        ], axis=1)
        h = x @ w1_ref[...] + b1_ref[...]
        h = jnp.where(h >= 0, h, 0.01 * h)
        h = h @ w2_ref[...] + b2_ref[...]
        h = jnp.where(h >= 0, h, 0.01 * h)
        o_ref[...] = h @ w3_ref[...] + b3_ref[...]

    return pl.pallas_call(
        body,
        grid=(_B // blk,),
        in_specs=[
            pl.BlockSpec((blk, _DD), lambda i: (i, 0)),
            pl.BlockSpec((blk, _DP), lambda i: (i, 0)),
            pl.BlockSpec((blk, _DD), lambda i: (i, 0)),
            pl.BlockSpec((blk, _DP), lambda i: (i, 0)),
            pl.BlockSpec((blk, 32), lambda i: (i, 0)),
            pl.BlockSpec((128, 128), lambda i: (0, 0)),
            pl.BlockSpec((1, 128), lambda i: (0, 0)),
            pl.BlockSpec((128, 64), lambda i: (0, 0)),
            pl.BlockSpec((1, 64), lambda i: (0, 0)),
            pl.BlockSpec((64, 1), lambda i: (0, 0)),
            pl.BlockSpec((1, 1), lambda i: (0, 0)),
        ],
        out_specs=pl.BlockSpec((blk, 1), lambda i: (i, 0)),
        out_shape=jax.ShapeDtypeStruct((_B, 1), jnp.float32),
    )(cd_s, cp_s, pd_s, pp_s, sub,
      fc1_W, fc1_b.reshape(1, 128), fc2_W, fc2_b.reshape(1, 64),
      fc3_W, fc3_b.reshape(1, 1))


def kernel(compound_diseases, compound_phenotypes,
           compound_subcellular_locations, protein_diseases,
           protein_phenotypes, protein_subcellular_locations,
           disease_table, phenotype_table, subcellular_table,
           dW1, db1, dW2, db2, pW1, pb1, pW2, pb2, sW1, sb1, sW2, sb2,
           fc1_W, fc1_b, fc2_W, fc2_b, fc3_W, fc3_b):
    dis_aug = _augment(disease_table, dW1, db1, dW2)
    cd_s, pd_s = _sc_pool(dis_aug, compound_diseases, protein_diseases, 3)

    phe_aug = _augment(phenotype_table, pW1, pb1, pW2)
    cp_s, pp_s = _sc_pool(phe_aug, compound_phenotypes, protein_phenotypes, 2)

    st_pad = jnp.pad(subcellular_table, ((0, 2), (0, 0)))
    sub = _sub_pool(compound_subcellular_locations,
                    protein_subcellular_locations, st_pad, sW1, sb1, sW2)

    return _final(cd_s, cp_s, pd_s, pp_s, sub,
                  fc1_W, fc1_b, fc2_W, fc2_b, fc3_W, fc3_b)
